# baseline (speedup 1.0000x reference)
import functools

import jax
import jax.numpy as jnp
from jax import lax
from jax.experimental import pallas as pl
from jax.experimental.pallas import tpu as pltpu
from jax.experimental.pallas import tpu_sc as plsc

N, E, D, H = 10000, 320000, 128, 16
NC, NS = 2, 16
NW = NC * NS
EV = E // NW
GROUP = 400
G = EV // GROUP
NBUF = 3
NSLOT = 2 * NBUF
GLOOP = 24
NPAD = 10112
ZR = NPAD // NS


N8 = N // 8
NPAD8 = NPAD // 8


def _mm1_body(x_ref, w_ref, o_ref):
    o_ref[...] = jnp.concatenate(
        [jnp.dot(x_ref[:, j, :], w_ref[...],
                 preferred_element_type=jnp.float32,
                 precision=jax.lax.Precision.HIGHEST) for j in range(8)],
        axis=1)


def _mlp2_body(y_ref, p_ref, w2_ref, b1_ref, b2_ref, eps_ref, o_ref):
    b1t = jnp.concatenate([b1_ref[...]] * 8, axis=1)
    b2t = jnp.concatenate([b2_ref[...]] * 8, axis=1)
    h = ((1.0 + eps_ref[...]) * y_ref[...]
         + p_ref[0, :N8] + p_ref[1, :N8] + b1t)
    h = jnp.maximum(h, 0.0)
    h = jnp.concatenate(
        [jnp.dot(h[:, 16 * j:16 * (j + 1)], w2_ref[...],
                 preferred_element_type=jnp.float32,
                 precision=jax.lax.Precision.HIGHEST) for j in range(8)],
        axis=1) + b2t
    o_ref[...] = jnp.maximum(h, 0.0)


@functools.partial(
    pl.kernel,
    mesh=plsc.VectorSubcoreMesh(core_axis_name="c", subcore_axis_name="s"),
    out_type=jax.ShapeDtypeStruct((NC, NPAD8, 8 * H), jnp.float32),
    compiler_params=pltpu.CompilerParams(use_tc_tiling_on_sc=False),
    scratch_types=[
        pltpu.VMEM((G, GROUP), jnp.int32),
        pltpu.VMEM((G, GROUP), jnp.int32),
        pltpu.VMEM((NSLOT, GROUP, H), jnp.float32),
        pltpu.VMEM((125, 8 * H), jnp.float32),
        pltpu.VMEM((1000, H), jnp.float32),
        pltpu.VMEM_SHARED((NPAD, H), jnp.float32),
        pltpu.VMEM_SHARED((N, H), jnp.float32),
        pltpu.SemaphoreType.DMA((NSLOT,)),
        pltpu.SemaphoreType.DMA((NSLOT,)),
        pltpu.SemaphoreType.DMA,
    ],
)
def _sc_scatter(y_hbm, ei_hbm, zero_hbm, out_hbm,
                src_v, dst_v, rows_v, t128_v, t16_v, agg_sh, y_sh,
                sem_g, sem_s, sem_i):
    cid = lax.axis_index("c")
    sid = lax.axis_index("s")
    wid = sid * NC + cid
    ebase = wid * EV
    for g in range(G):
        pltpu.async_copy(ei_hbm.at[0, pl.ds(ebase + g * GROUP, GROUP)],
                         src_v.at[g], sem_i)
        pltpu.async_copy(ei_hbm.at[1, pl.ds(ebase + g * GROUP, GROUP)],
                         dst_v.at[g], sem_i)
    pltpu.sync_copy(zero_hbm.at[pl.ds(sid * ZR, ZR)],
                    agg_sh.at[pl.ds(sid * ZR, ZR)])

    @pl.when(sid < 10)
    def _():
        pltpu.sync_copy(y_hbm.at[pl.ds(sid * 125, 125)], t128_v)

        def repack(r, carry):
            for j in range(8):
                t16_v[8 * r + j, :] = t128_v[r, 16 * j:16 * (j + 1)]
            return carry

        lax.fori_loop(0, 125, repack, 0)
        pltpu.sync_copy(t16_v, y_sh.at[pl.ds(sid * 1000, 1000)])

    for g in range(G):
        pltpu.make_async_copy(ei_hbm.at[0, pl.ds(ebase + g * GROUP, GROUP)],
                              src_v.at[g], sem_i).wait()
        pltpu.make_async_copy(ei_hbm.at[1, pl.ds(ebase + g * GROUP, GROUP)],
                              dst_v.at[g], sem_i).wait()
    plsc.subcore_barrier()

    def body(it, carry):
        for p in range(2):
            base = (2 * it + p) * NBUF
            for b in range(NBUF):
                slot = p * NBUF + b

                @pl.when(it > 0)
                def _():
                    pltpu.make_async_copy(
                        rows_v.at[slot], agg_sh.at[dst_v.at[base + b]],
                        sem_s.at[slot]).wait()

                pltpu.async_copy(y_sh.at[src_v.at[base + b]],
                                 rows_v.at[slot], sem_g.at[slot])
            for b in range(NBUF):
                slot = p * NBUF + b
                pltpu.make_async_copy(y_sh.at[src_v.at[base + b]],
                                      rows_v.at[slot], sem_g.at[slot]).wait()
                pltpu.async_copy(rows_v.at[slot],
                                 agg_sh.at[dst_v.at[base + b]],
                                 sem_s.at[slot], add=True)
        return carry

    lax.fori_loop(0, GLOOP // (2 * NBUF), body, 0)
    for slot in range(NSLOT):
        g_last = GLOOP - NSLOT + slot
        pltpu.make_async_copy(rows_v.at[slot], agg_sh.at[dst_v.at[g_last]],
                              sem_s.at[slot]).wait()
    pltpu.async_copy(y_sh.at[src_v.at[GLOOP]], rows_v.at[0],
                     sem_g.at[0]).wait()
    pltpu.sync_copy(rows_v.at[0], agg_sh.at[dst_v.at[GLOOP]], add=True)
    plsc.subcore_barrier()
    pltpu.sync_copy(agg_sh.at[pl.ds(sid * ZR, ZR)], t16_v.at[pl.ds(0, ZR)])

    def repack_out(r, carry):
        for j in range(8):
            t128_v[r, 16 * j:16 * (j + 1)] = t16_v[8 * r + j, :]
        return carry

    lax.fori_loop(0, ZR // 8, repack_out, 0)
    pltpu.sync_copy(t128_v.at[pl.ds(0, ZR // 8)],
                    out_hbm.at[cid, pl.ds(sid * (ZR // 8), ZR // 8)])


def kernel(x, edge_index, W1, b1, W2, b2, eps):
    x3 = x.reshape(N8, 8, D)
    y8 = pl.pallas_call(
        _mm1_body,
        out_shape=jax.ShapeDtypeStruct((N8, 8 * H), jnp.float32),
    )(x3, W1)

    zeros = jnp.zeros((NPAD, H), jnp.float32)
    parts = _sc_scatter(y8, edge_index, zeros)

    out8 = pl.pallas_call(
        _mlp2_body,
        out_shape=jax.ShapeDtypeStruct((N8, 8 * H), jnp.float32),
    )(y8, parts, W2, b1.reshape(1, H), b2.reshape(1, H), eps.reshape(1, 1))
    return out8.reshape(N, H)


# speedup vs baseline: 1.1229x; 1.1229x over previous
import functools

import jax
import jax.numpy as jnp
from jax import lax
from jax.experimental import pallas as pl
from jax.experimental.pallas import tpu as pltpu
from jax.experimental.pallas import tpu_sc as plsc

N, E, D, H = 10000, 320000, 128, 16
NC, NS = 2, 16
NW = NC * NS
EV = E // NW
GROUP = 400
G = EV // GROUP
NBUF = 3
NSLOT = 2 * NBUF
GLOOP = 24
NPAD = 10112
ZR = NPAD // NS


N8 = N // 8
NPAD8 = NPAD // 8


def _blockdiag8(w_ref, k):
    return jnp.concatenate(
        [jnp.pad(w_ref[...], ((0, 0), (16 * j, 112 - 16 * j)))
         for j in range(8)], axis=0)


def _mm1_body(x_ref, w_ref, o_ref):
    o_ref[...] = jnp.dot(x_ref[...], _blockdiag8(w_ref, D),
                         preferred_element_type=jnp.float32,
                         precision=jax.lax.Precision.HIGHEST)


def _mlp2_body(y_ref, p_ref, w2_ref, b1_ref, b2_ref, eps_ref, o_ref):
    b1t = jnp.concatenate([b1_ref[...]] * 8, axis=1)
    b2t = jnp.concatenate([b2_ref[...]] * 8, axis=1)
    h = ((1.0 + eps_ref[...]) * y_ref[...]
         + p_ref[0, :N8] + p_ref[1, :N8] + b1t)
    h = jnp.maximum(h, 0.0)
    h = jnp.dot(h, _blockdiag8(w2_ref, H), preferred_element_type=jnp.float32,
                precision=jax.lax.Precision.HIGHEST) + b2t
    o_ref[...] = jnp.maximum(h, 0.0)


@functools.partial(
    pl.kernel,
    mesh=plsc.VectorSubcoreMesh(core_axis_name="c", subcore_axis_name="s"),
    out_type=jax.ShapeDtypeStruct((NC, NPAD8, 8 * H), jnp.float32),
    compiler_params=pltpu.CompilerParams(use_tc_tiling_on_sc=False),
    scratch_types=[
        pltpu.VMEM((G, GROUP), jnp.int32),
        pltpu.VMEM((G, GROUP), jnp.int32),
        pltpu.VMEM((NSLOT, GROUP, H), jnp.float32),
        pltpu.VMEM((125, 8 * H), jnp.float32),
        pltpu.VMEM((1000, H), jnp.float32),
        pltpu.VMEM_SHARED((NPAD, H), jnp.float32),
        pltpu.VMEM_SHARED((N, H), jnp.float32),
        pltpu.SemaphoreType.DMA((NSLOT,)),
        pltpu.SemaphoreType.DMA((NSLOT,)),
        pltpu.SemaphoreType.DMA,
    ],
)
def _sc_scatter(y_hbm, ei_hbm, zero_hbm, out_hbm,
                src_v, dst_v, rows_v, t128_v, t16_v, agg_sh, y_sh,
                sem_g, sem_s, sem_i):
    cid = lax.axis_index("c")
    sid = lax.axis_index("s")
    wid = sid * NC + cid
    ebase = wid * EV
    for g in range(G):
        pltpu.async_copy(ei_hbm.at[0, pl.ds(ebase + g * GROUP, GROUP)],
                         src_v.at[g], sem_i)
        pltpu.async_copy(ei_hbm.at[1, pl.ds(ebase + g * GROUP, GROUP)],
                         dst_v.at[g], sem_i)
    pltpu.sync_copy(zero_hbm.at[pl.ds(sid * ZR, ZR)],
                    agg_sh.at[pl.ds(sid * ZR, ZR)])

    @pl.when(sid < 10)
    def _():
        pltpu.sync_copy(y_hbm.at[pl.ds(sid * 125, 125)], t128_v)

        def repack(r, carry):
            for j in range(8):
                t16_v[8 * r + j, :] = t128_v[r, 16 * j:16 * (j + 1)]
            return carry

        lax.fori_loop(0, 125, repack, 0)
        pltpu.sync_copy(t16_v, y_sh.at[pl.ds(sid * 1000, 1000)])

    for g in range(G):
        pltpu.make_async_copy(ei_hbm.at[0, pl.ds(ebase + g * GROUP, GROUP)],
                              src_v.at[g], sem_i).wait()
        pltpu.make_async_copy(ei_hbm.at[1, pl.ds(ebase + g * GROUP, GROUP)],
                              dst_v.at[g], sem_i).wait()
    plsc.subcore_barrier()

    def body(it, carry):
        for p in range(2):
            base = (2 * it + p) * NBUF
            for b in range(NBUF):
                slot = p * NBUF + b

                @pl.when(it > 0)
                def _():
                    pltpu.make_async_copy(
                        rows_v.at[slot], agg_sh.at[dst_v.at[base + b]],
                        sem_s.at[slot]).wait()

                pltpu.async_copy(y_sh.at[src_v.at[base + b]],
                                 rows_v.at[slot], sem_g.at[slot])
            for b in range(NBUF):
                slot = p * NBUF + b
                pltpu.make_async_copy(y_sh.at[src_v.at[base + b]],
                                      rows_v.at[slot], sem_g.at[slot]).wait()
                pltpu.async_copy(rows_v.at[slot],
                                 agg_sh.at[dst_v.at[base + b]],
                                 sem_s.at[slot], add=True)
        return carry

    lax.fori_loop(0, GLOOP // (2 * NBUF), body, 0)
    for slot in range(NSLOT):
        g_last = GLOOP - NSLOT + slot
        pltpu.make_async_copy(rows_v.at[slot], agg_sh.at[dst_v.at[g_last]],
                              sem_s.at[slot]).wait()
    pltpu.async_copy(y_sh.at[src_v.at[GLOOP]], rows_v.at[0],
                     sem_g.at[0]).wait()
    pltpu.sync_copy(rows_v.at[0], agg_sh.at[dst_v.at[GLOOP]], add=True)
    plsc.subcore_barrier()
    pltpu.sync_copy(agg_sh.at[pl.ds(sid * ZR, ZR)], t16_v.at[pl.ds(0, ZR)])

    def repack_out(r, carry):
        for j in range(8):
            t128_v[r, 16 * j:16 * (j + 1)] = t16_v[8 * r + j, :]
        return carry

    lax.fori_loop(0, ZR // 8, repack_out, 0)
    pltpu.sync_copy(t128_v.at[pl.ds(0, ZR // 8)],
                    out_hbm.at[cid, pl.ds(sid * (ZR // 8), ZR // 8)])


def kernel(x, edge_index, W1, b1, W2, b2, eps):
    x8 = x.reshape(N8, 8 * D)
    y8 = pl.pallas_call(
        _mm1_body,
        out_shape=jax.ShapeDtypeStruct((N8, 8 * H), jnp.float32),
    )(x8, W1)

    zeros = jnp.zeros((NPAD, H), jnp.float32)
    parts = _sc_scatter(y8, edge_index, zeros)

    out8 = pl.pallas_call(
        _mlp2_body,
        out_shape=jax.ShapeDtypeStruct((N8, 8 * H), jnp.float32),
    )(y8, parts, W2, b1.reshape(1, H), b2.reshape(1, H), eps.reshape(1, 1))
    return out8.reshape(N, H)


# speedup vs baseline: 1.1437x; 1.0185x over previous
import functools

import jax
import jax.numpy as jnp
from jax import lax
from jax.experimental import pallas as pl
from jax.experimental.pallas import tpu as pltpu
from jax.experimental.pallas import tpu_sc as plsc

N, E, D, H = 10000, 320000, 128, 16
NC, NS = 2, 16
NW = NC * NS
EV = E // NW
GROUP = 400
G = EV // GROUP
NBUF = 4
NSLOT = 2 * NBUF
GLOOP = 24
NPAD = 10112
ZR = NPAD // NS


N8 = N // 8
NPAD8 = NPAD // 8


def _blockdiag8(w_ref, k):
    return jnp.concatenate(
        [jnp.pad(w_ref[...], ((0, 0), (16 * j, 112 - 16 * j)))
         for j in range(8)], axis=0)


def _mm1_body(x_ref, w_ref, o_ref):
    o_ref[...] = jnp.dot(x_ref[...], _blockdiag8(w_ref, D),
                         preferred_element_type=jnp.float32,
                         precision=jax.lax.Precision.HIGHEST)


def _mlp2_body(y_ref, p_ref, w2_ref, b1_ref, b2_ref, eps_ref, o_ref):
    b1t = jnp.concatenate([b1_ref[...]] * 8, axis=1)
    b2t = jnp.concatenate([b2_ref[...]] * 8, axis=1)
    h = ((1.0 + eps_ref[...]) * y_ref[...]
         + p_ref[0, :N8] + p_ref[1, :N8] + b1t)
    h = jnp.maximum(h, 0.0)
    h = jnp.dot(h, _blockdiag8(w2_ref, H), preferred_element_type=jnp.float32,
                precision=jax.lax.Precision.HIGHEST) + b2t
    o_ref[...] = jnp.maximum(h, 0.0)


@functools.partial(
    pl.kernel,
    mesh=plsc.VectorSubcoreMesh(core_axis_name="c", subcore_axis_name="s"),
    out_type=jax.ShapeDtypeStruct((NC, NPAD8, 8 * H), jnp.float32),
    compiler_params=pltpu.CompilerParams(use_tc_tiling_on_sc=False),
    scratch_types=[
        pltpu.VMEM((G, GROUP), jnp.int32),
        pltpu.VMEM((G, GROUP), jnp.int32),
        pltpu.VMEM((NSLOT, GROUP, H), jnp.float32),
        pltpu.VMEM((125, 8 * H), jnp.float32),
        pltpu.VMEM((1000, H), jnp.float32),
        pltpu.VMEM_SHARED((NPAD, H), jnp.float32),
        pltpu.VMEM_SHARED((N, H), jnp.float32),
        pltpu.SemaphoreType.DMA((NSLOT,)),
        pltpu.SemaphoreType.DMA((NSLOT,)),
        pltpu.SemaphoreType.DMA,
    ],
)
def _sc_scatter(y_hbm, ei_hbm, zero_hbm, out_hbm,
                src_v, dst_v, rows_v, t128_v, t16_v, agg_sh, y_sh,
                sem_g, sem_s, sem_i):
    cid = lax.axis_index("c")
    sid = lax.axis_index("s")
    wid = sid * NC + cid
    ebase = wid * EV
    for g in range(G):
        pltpu.async_copy(ei_hbm.at[0, pl.ds(ebase + g * GROUP, GROUP)],
                         src_v.at[g], sem_i)
        pltpu.async_copy(ei_hbm.at[1, pl.ds(ebase + g * GROUP, GROUP)],
                         dst_v.at[g], sem_i)
    pltpu.sync_copy(zero_hbm.at[pl.ds(sid * ZR, ZR)],
                    agg_sh.at[pl.ds(sid * ZR, ZR)])

    @pl.when(sid < 10)
    def _():
        pltpu.sync_copy(y_hbm.at[pl.ds(sid * 125, 125)], t128_v)

        def repack(r5, carry):
            for i in range(5):
                for j in range(8):
                    r = 5 * r5 + i
                    t16_v[8 * r + j, :] = t128_v[r, 16 * j:16 * (j + 1)]
            return carry

        lax.fori_loop(0, 25, repack, 0)
        pltpu.sync_copy(t16_v, y_sh.at[pl.ds(sid * 1000, 1000)])

    for g in range(G):
        pltpu.make_async_copy(ei_hbm.at[0, pl.ds(ebase + g * GROUP, GROUP)],
                              src_v.at[g], sem_i).wait()
        pltpu.make_async_copy(ei_hbm.at[1, pl.ds(ebase + g * GROUP, GROUP)],
                              dst_v.at[g], sem_i).wait()
    plsc.subcore_barrier()

    def body(it, carry):
        for p in range(2):
            base = (2 * it + p) * NBUF
            for b in range(NBUF):
                slot = p * NBUF + b

                @pl.when(it > 0)
                def _():
                    pltpu.make_async_copy(
                        rows_v.at[slot], agg_sh.at[dst_v.at[base + b]],
                        sem_s.at[slot]).wait()

                pltpu.async_copy(y_sh.at[src_v.at[base + b]],
                                 rows_v.at[slot], sem_g.at[slot])
            for b in range(NBUF):
                slot = p * NBUF + b
                pltpu.make_async_copy(y_sh.at[src_v.at[base + b]],
                                      rows_v.at[slot], sem_g.at[slot]).wait()
                pltpu.async_copy(rows_v.at[slot],
                                 agg_sh.at[dst_v.at[base + b]],
                                 sem_s.at[slot], add=True)
        return carry

    lax.fori_loop(0, GLOOP // (2 * NBUF), body, 0)
    for slot in range(NSLOT):
        g_last = GLOOP - NSLOT + slot
        pltpu.make_async_copy(rows_v.at[slot], agg_sh.at[dst_v.at[g_last]],
                              sem_s.at[slot]).wait()
    pltpu.async_copy(y_sh.at[src_v.at[GLOOP]], rows_v.at[0],
                     sem_g.at[0]).wait()
    pltpu.sync_copy(rows_v.at[0], agg_sh.at[dst_v.at[GLOOP]], add=True)
    plsc.subcore_barrier()
    pltpu.sync_copy(agg_sh.at[pl.ds(sid * ZR, ZR)], t16_v.at[pl.ds(0, ZR)])

    def repack_out(r, carry):
        for j in range(8):
            t128_v[r, 16 * j:16 * (j + 1)] = t16_v[8 * r + j, :]
        return carry

    lax.fori_loop(0, ZR // 8, repack_out, 0)
    pltpu.sync_copy(t128_v.at[pl.ds(0, ZR // 8)],
                    out_hbm.at[cid, pl.ds(sid * (ZR // 8), ZR // 8)])


def kernel(x, edge_index, W1, b1, W2, b2, eps):
    x8 = x.reshape(N8, 8 * D)
    y8 = pl.pallas_call(
        _mm1_body,
        out_shape=jax.ShapeDtypeStruct((N8, 8 * H), jnp.float32),
    )(x8, W1)

    zeros = jnp.zeros((NPAD, H), jnp.float32)
    parts = _sc_scatter(y8, edge_index, zeros)

    out8 = pl.pallas_call(
        _mlp2_body,
        out_shape=jax.ShapeDtypeStruct((N8, 8 * H), jnp.float32),
    )(y8, parts, W2, b1.reshape(1, H), b2.reshape(1, H), eps.reshape(1, 1))
    return out8.reshape(N, H)


# speedup vs baseline: 1.1609x; 1.0151x over previous
import functools

import jax
import jax.numpy as jnp
from jax import lax
from jax.experimental import pallas as pl
from jax.experimental.pallas import tpu as pltpu
from jax.experimental.pallas import tpu_sc as plsc

N, E, D, H = 10000, 320000, 128, 16
NC, NS = 2, 16
NW = NC * NS
EV = E // NW
GROUP = 400
G = EV // GROUP
NBUF = 4
NSLOT = 2 * NBUF
GLOOP = 24
NPAD = 10112
ZR = NPAD // NS


N8 = N // 8
NPAD8 = NPAD // 8


def _blockdiag8(w_ref, k):
    return jnp.concatenate(
        [jnp.pad(w_ref[...], ((0, 0), (16 * j, 112 - 16 * j)))
         for j in range(8)], axis=0)


def _mm1_body(x_ref, w_ref, o_ref):
    o_ref[...] = jnp.dot(x_ref[...], _blockdiag8(w_ref, D),
                         preferred_element_type=jnp.float32,
                         precision=jax.lax.Precision.HIGHEST)


def _mlp2_body(y_ref, p_ref, w2_ref, b1_ref, b2_ref, eps_ref, o_ref):
    b1t = jnp.concatenate([b1_ref[...]] * 8, axis=1)
    b2t = jnp.concatenate([b2_ref[...]] * 8, axis=1)
    h = ((1.0 + eps_ref[...]) * y_ref[...]
         + p_ref[0, :N8] + p_ref[1, :N8] + b1t)
    h = jnp.maximum(h, 0.0)
    h = jnp.dot(h, _blockdiag8(w2_ref, H), preferred_element_type=jnp.float32,
                precision=jax.lax.Precision.HIGHEST) + b2t
    o_ref[...] = jnp.maximum(h, 0.0)


@functools.partial(
    pl.kernel,
    mesh=plsc.VectorSubcoreMesh(core_axis_name="c", subcore_axis_name="s"),
    out_type=jax.ShapeDtypeStruct((NC, NPAD8, 8 * H), jnp.float32),
    compiler_params=pltpu.CompilerParams(use_tc_tiling_on_sc=False),
    scratch_types=[
        pltpu.VMEM((G, GROUP), jnp.int32),
        pltpu.VMEM((G, GROUP), jnp.int32),
        pltpu.VMEM((NSLOT, GROUP, H), jnp.float32),
        pltpu.VMEM((125, 8 * H), jnp.float32),
        pltpu.VMEM((1000, H), jnp.float32),
        pltpu.VMEM_SHARED((NPAD, H), jnp.float32),
        pltpu.VMEM_SHARED((N, H), jnp.float32),
        pltpu.SemaphoreType.DMA((NSLOT,)),
        pltpu.SemaphoreType.DMA((NSLOT,)),
        pltpu.SemaphoreType.DMA,
    ],
)
def _sc_scatter(y_hbm, ei_hbm, zero_hbm, out_hbm,
                src_v, dst_v, rows_v, t128_v, t16_v, agg_sh, y_sh,
                sem_g, sem_s, sem_i):
    cid = lax.axis_index("c")
    sid = lax.axis_index("s")
    wid = sid * NC + cid
    ebase = wid * EV
    for g in range(G):
        pltpu.async_copy(ei_hbm.at[0, pl.ds(ebase + g * GROUP, GROUP)],
                         src_v.at[g], sem_i)
        pltpu.async_copy(ei_hbm.at[1, pl.ds(ebase + g * GROUP, GROUP)],
                         dst_v.at[g], sem_i)
    pltpu.async_copy(zero_hbm.at[pl.ds(sid * ZR, ZR)],
                     agg_sh.at[pl.ds(sid * ZR, ZR)], sem_i)

    @pl.when(sid < 10)
    def _():
        pltpu.async_copy(y_hbm.at[pl.ds(sid * 125, 125)], t128_v, sem_i)
        pltpu.make_async_copy(y_hbm.at[pl.ds(sid * 125, 125)], t128_v,
                              sem_i).wait()

        def repack(r5, carry):
            for i in range(5):
                for j in range(8):
                    r = 5 * r5 + i
                    t16_v[8 * r + j, :] = t128_v[r, 16 * j:16 * (j + 1)]
            return carry

        lax.fori_loop(0, 25, repack, 0)
        pltpu.async_copy(t16_v, y_sh.at[pl.ds(sid * 1000, 1000)], sem_i)
        pltpu.make_async_copy(t16_v, y_sh.at[pl.ds(sid * 1000, 1000)],
                              sem_i).wait()

    for g in range(G):
        pltpu.make_async_copy(ei_hbm.at[0, pl.ds(ebase + g * GROUP, GROUP)],
                              src_v.at[g], sem_i).wait()
        pltpu.make_async_copy(ei_hbm.at[1, pl.ds(ebase + g * GROUP, GROUP)],
                              dst_v.at[g], sem_i).wait()
    pltpu.make_async_copy(zero_hbm.at[pl.ds(sid * ZR, ZR)],
                          agg_sh.at[pl.ds(sid * ZR, ZR)], sem_i).wait()
    plsc.subcore_barrier()

    def body(it, carry):
        for p in range(2):
            base = (2 * it + p) * NBUF
            for b in range(NBUF):
                slot = p * NBUF + b

                @pl.when(it > 0)
                def _():
                    pltpu.make_async_copy(
                        rows_v.at[slot], agg_sh.at[dst_v.at[base + b]],
                        sem_s.at[slot]).wait()

                pltpu.async_copy(y_sh.at[src_v.at[base + b]],
                                 rows_v.at[slot], sem_g.at[slot])
            for b in range(NBUF):
                slot = p * NBUF + b
                pltpu.make_async_copy(y_sh.at[src_v.at[base + b]],
                                      rows_v.at[slot], sem_g.at[slot]).wait()
                pltpu.async_copy(rows_v.at[slot],
                                 agg_sh.at[dst_v.at[base + b]],
                                 sem_s.at[slot], add=True)
        return carry

    lax.fori_loop(0, GLOOP // (2 * NBUF), body, 0)
    for slot in range(NSLOT):
        g_last = GLOOP - NSLOT + slot
        pltpu.make_async_copy(rows_v.at[slot], agg_sh.at[dst_v.at[g_last]],
                              sem_s.at[slot]).wait()
    pltpu.async_copy(y_sh.at[src_v.at[GLOOP]], rows_v.at[0],
                     sem_g.at[0]).wait()
    pltpu.sync_copy(rows_v.at[0], agg_sh.at[dst_v.at[GLOOP]], add=True)
    plsc.subcore_barrier()
    pltpu.sync_copy(agg_sh.at[pl.ds(sid * ZR, ZR)], t16_v.at[pl.ds(0, ZR)])

    def repack_out(r, carry):
        for j in range(8):
            t128_v[r, 16 * j:16 * (j + 1)] = t16_v[8 * r + j, :]
        return carry

    lax.fori_loop(0, ZR // 8, repack_out, 0)
    pltpu.sync_copy(t128_v.at[pl.ds(0, ZR // 8)],
                    out_hbm.at[cid, pl.ds(sid * (ZR // 8), ZR // 8)])


def kernel(x, edge_index, W1, b1, W2, b2, eps):
    x8 = x.reshape(N8, 8 * D)
    y8 = pl.pallas_call(
        _mm1_body,
        out_shape=jax.ShapeDtypeStruct((N8, 8 * H), jnp.float32),
    )(x8, W1)

    zeros = jnp.zeros((NPAD, H), jnp.float32)
    parts = _sc_scatter(y8, edge_index, zeros)

    out8 = pl.pallas_call(
        _mlp2_body,
        out_shape=jax.ShapeDtypeStruct((N8, 8 * H), jnp.float32),
    )(y8, parts, W2, b1.reshape(1, H), b2.reshape(1, H), eps.reshape(1, 1))
    return out8.reshape(N, H)
